# Initial kernel scaffold; baseline (speedup 1.0000x reference)
import jax
import jax.numpy as jnp
from jax.experimental import pallas as pl


def kernel(x, edge_index, W1, b1, W2, b2, W3, b3):
    raise NotImplementedError("write your pallas kernel here")


# speedup vs baseline: 8.2482x; 8.2482x over previous
import functools

import jax
import jax.numpy as jnp
from jax import lax
from jax.experimental import pallas as pl
from jax.experimental.pallas import tpu as pltpu
from jax.experimental.pallas import tpu_sc as plsc

N = 10000
E = 320000
D = 128
NC = 2
NS = 16
NT = NC * NS
B = 128
K = 80
KH = K // 2
E_PAD = NT * K * B
N_PAD = 10112
RPT = N_PAD // NS
R = 2528
G = N_PAD // R

_MESH = plsc.VectorSubcoreMesh(
    core_axis_name="c", subcore_axis_name="s", num_cores=NC, num_subcores=NS
)


def _sc_deg_body(dst_hbm, ones_hbm, zeros16_hbm, degp_hbm, dst_v, ones_v, deg_sh):
    c = lax.axis_index("c")
    s = lax.axis_index("s")
    wid = c * NS + s
    pltpu.sync_copy(dst_hbm.at[wid], dst_v)
    pltpu.sync_copy(ones_hbm, ones_v)
    pltpu.sync_copy(zeros16_hbm, deg_sh.at[pl.ds(s * RPT, RPT)])
    plsc.subcore_barrier()

    def estep(j, carry):
        pltpu.sync_copy(ones_v, deg_sh.at[dst_v.at[j]], add=True)
        return carry

    lax.fori_loop(0, K, estep, 0)
    plsc.subcore_barrier()
    pltpu.sync_copy(deg_sh.at[pl.ds(s * RPT, RPT)],
                    degp_hbm.at[c, pl.ds(s * RPT, RPT)])


_sc_deg = pl.kernel(
    _sc_deg_body,
    out_type=jax.ShapeDtypeStruct((NC, N_PAD, 16), jnp.float32),
    mesh=_MESH,
    scratch_types=[
        pltpu.VMEM((K, B), jnp.int32),
        pltpu.VMEM((B, 16), jnp.float32),
        pltpu.VMEM_SHARED((N_PAD, 16), jnp.float32),
    ],
)


def _sc_layer_body(g_hbm, src_hbm, dst_hbm, zeros_hbm, p_hbm,
                   src_v, dst_v, bufa, bufb, acc, sema, semb):
    c = lax.axis_index("c")
    s = lax.axis_index("s")
    wid = c * NS + s
    pltpu.sync_copy(zeros_hbm, acc.at[pl.ds(s * RPT, RPT)])
    plsc.subcore_barrier()

    for h in range(2):
        pltpu.sync_copy(src_hbm.at[wid, pl.ds(h * KH, KH)], src_v)
        pltpu.sync_copy(dst_hbm.at[wid, pl.ds(h * KH, KH)], dst_v)
        pltpu.async_copy(g_hbm.at[src_v.at[0]], bufa, sema)

        def step(i, carry):
            j = 2 * i
            pltpu.make_async_copy(g_hbm.at[src_v.at[j]], bufa, sema).wait()
            pltpu.async_copy(g_hbm.at[src_v.at[j + 1]], bufb, semb)
            pltpu.sync_copy(bufa, acc.at[dst_v.at[j]], add=True)
            pltpu.make_async_copy(g_hbm.at[src_v.at[j + 1]], bufb, semb).wait()

            @pl.when(j + 2 < KH)
            def _():
                pltpu.async_copy(g_hbm.at[src_v.at[j + 2]], bufa, sema)

            pltpu.sync_copy(bufb, acc.at[dst_v.at[j + 1]], add=True)
            return carry

        lax.fori_loop(0, KH // 2, step, 0)
    plsc.subcore_barrier()
    pltpu.sync_copy(acc.at[pl.ds(s * RPT, RPT)], p_hbm.at[c, pl.ds(s * RPT, RPT)])


_sc_layer = pl.kernel(
    _sc_layer_body,
    out_type=jax.ShapeDtypeStruct((NC, N_PAD, D), jnp.float32),
    mesh=_MESH,
    scratch_types=[
        pltpu.VMEM((KH, B), jnp.int32),
        pltpu.VMEM((KH, B), jnp.int32),
        pltpu.VMEM((B, D), jnp.float32),
        pltpu.VMEM((B, D), jnp.float32),
        pltpu.VMEM_SHARED((N_PAD, D), jnp.float32),
        pltpu.SemaphoreType.DMA,
        pltpu.SemaphoreType.DMA,
    ],
)


def _tc_first_body(x_ref, w_ref, pt_ref, g_ref, dis_ref):
    deg = pt_ref[0, :, 0:1] + pt_ref[1, :, 0:1] + 1.0
    dv = lax.rsqrt(deg)
    dis_ref[...] = dv
    g_ref[...] = dv * jnp.dot(x_ref[...], w_ref[...],
                              preferred_element_type=jnp.float32)


_tc_first = pl.pallas_call(
    _tc_first_body,
    grid=(G,),
    in_specs=[
        pl.BlockSpec((R, D), lambda i: (i, 0)),
        pl.BlockSpec((D, D), lambda i: (0, 0)),
        pl.BlockSpec((NC, R, 16), lambda i: (0, i, 0)),
    ],
    out_specs=[
        pl.BlockSpec((R, D), lambda i: (i, 0)),
        pl.BlockSpec((R, 1), lambda i: (i, 0)),
    ],
    out_shape=[
        jax.ShapeDtypeStruct((N_PAD, D), jnp.float32),
        jax.ShapeDtypeStruct((N_PAD, 1), jnp.float32),
    ],
)


def _tc_mid_body(p_ref, g_ref, dis_ref, w_ref, b_ref, o_ref):
    dv = dis_ref[...]
    ssum = p_ref[0] + p_ref[1] + g_ref[...]
    a = jnp.maximum(dv * ssum + b_ref[...], 0.0)
    o_ref[...] = dv * jnp.dot(a, w_ref[...], preferred_element_type=jnp.float32)


_tc_mid = pl.pallas_call(
    _tc_mid_body,
    grid=(G,),
    in_specs=[
        pl.BlockSpec((NC, R, D), lambda i: (0, i, 0)),
        pl.BlockSpec((R, D), lambda i: (i, 0)),
        pl.BlockSpec((R, 1), lambda i: (i, 0)),
        pl.BlockSpec((D, D), lambda i: (0, 0)),
        pl.BlockSpec((1, D), lambda i: (0, 0)),
    ],
    out_specs=pl.BlockSpec((R, D), lambda i: (i, 0)),
    out_shape=jax.ShapeDtypeStruct((N_PAD, D), jnp.float32),
)


def _tc_last_body(p_ref, g_ref, dis_ref, b_ref, o_ref):
    o_ref[...] = (dis_ref[...] * (p_ref[0] + p_ref[1] + g_ref[...])
                  + b_ref[...])


_tc_last = pl.pallas_call(
    _tc_last_body,
    grid=(G,),
    in_specs=[
        pl.BlockSpec((NC, R, D), lambda i: (0, i, 0)),
        pl.BlockSpec((R, D), lambda i: (i, 0)),
        pl.BlockSpec((R, 1), lambda i: (i, 0)),
        pl.BlockSpec((1, D), lambda i: (0, 0)),
    ],
    out_specs=pl.BlockSpec((R, D), lambda i: (i, 0)),
    out_shape=jax.ShapeDtypeStruct((N_PAD, D), jnp.float32),
)


def kernel(x, edge_index, W1, b1, W2, b2, W3, b3):
    src = edge_index[0].astype(jnp.int32)
    dst = edge_index[1].astype(jnp.int32)
    pad = jnp.full((E_PAD - E,), N, jnp.int32)
    src_t = jnp.concatenate([src, pad]).reshape(NT, K, B)
    dst_t = jnp.concatenate([dst, pad]).reshape(NT, K, B)
    x_pad = jnp.pad(x, ((0, N_PAD - N), (0, 0)))
    zeros_blk = jnp.zeros((RPT, D), jnp.float32)
    zeros16 = jnp.zeros((RPT, 16), jnp.float32)
    ones16 = jnp.ones((B, 16), jnp.float32)

    degp = _sc_deg(dst_t, ones16, zeros16)
    g1, dis = _tc_first(x_pad, W1, degp)
    p1 = _sc_layer(g1, src_t, dst_t, zeros_blk)
    g2 = _tc_mid(p1, g1, dis, W2, b1.reshape(1, D))
    p2 = _sc_layer(g2, src_t, dst_t, zeros_blk)
    g3 = _tc_mid(p2, g2, dis, W3, b2.reshape(1, D))
    p3 = _sc_layer(g3, src_t, dst_t, zeros_blk)
    out = _tc_last(p3, g3, dis, b3.reshape(1, D))
    return out[:N]


# speedup vs baseline: 23.9538x; 2.9041x over previous
import functools

import jax
import jax.numpy as jnp
from jax import lax
from jax.experimental import pallas as pl
from jax.experimental.pallas import tpu as pltpu
from jax.experimental.pallas import tpu_sc as plsc

N = 10000
E = 320000
D = 128
NC = 2
NS = 16
NT = NC * NS
B = 128
K = 80
KH = K // 2
E_PAD = NT * K * B
N_PAD = 10112
RPT = N_PAD // NS
R = 2528
G = N_PAD // R

_MESH = plsc.VectorSubcoreMesh(
    core_axis_name="c", subcore_axis_name="s", num_cores=NC, num_subcores=NS
)


def _sc_deg_body(dst_hbm, ones_hbm, zeros16_hbm, degp_hbm, dst_v, ones_v, deg_sh):
    c = lax.axis_index("c")
    s = lax.axis_index("s")
    wid = c * NS + s
    pltpu.sync_copy(dst_hbm.at[wid], dst_v)
    pltpu.sync_copy(ones_hbm, ones_v)
    pltpu.sync_copy(zeros16_hbm, deg_sh.at[pl.ds(s * RPT, RPT)])
    plsc.subcore_barrier()

    def estep(j, carry):
        pltpu.sync_copy(ones_v, deg_sh.at[dst_v.at[j]], add=True)
        return carry

    lax.fori_loop(0, K, estep, 0)
    plsc.subcore_barrier()
    pltpu.sync_copy(deg_sh.at[pl.ds(s * RPT, RPT)],
                    degp_hbm.at[c, pl.ds(s * RPT, RPT)])


_sc_deg = pl.kernel(
    _sc_deg_body,
    out_type=jax.ShapeDtypeStruct((NC, N_PAD, 16), jnp.float32),
    mesh=_MESH,
    scratch_types=[
        pltpu.VMEM((K, B), jnp.int32),
        pltpu.VMEM((B, 16), jnp.float32),
        pltpu.VMEM_SHARED((N_PAD, 16), jnp.float32),
    ],
)


def _sc_layer_body(g_hbm, src_hbm, dst_hbm, zeros_hbm, p_hbm,
                   src_v, dst_v, bufa, bufb, acc, sema, semb):
    c = lax.axis_index("c")
    s = lax.axis_index("s")
    wid = c * NS + s
    pltpu.sync_copy(zeros_hbm, acc.at[pl.ds(s * RPT, RPT)])
    plsc.subcore_barrier()

    for h in range(2):
        pltpu.sync_copy(src_hbm.at[wid, pl.ds(h * KH, KH)], src_v)
        pltpu.sync_copy(dst_hbm.at[wid, pl.ds(h * KH, KH)], dst_v)
        pltpu.async_copy(g_hbm.at[src_v.at[0]], bufa, sema)

        def step(i, carry):
            j = 2 * i
            pltpu.make_async_copy(g_hbm.at[src_v.at[j]], bufa, sema).wait()
            pltpu.async_copy(g_hbm.at[src_v.at[j + 1]], bufb, semb)
            pltpu.sync_copy(bufa, acc.at[dst_v.at[j]], add=True)
            pltpu.make_async_copy(g_hbm.at[src_v.at[j + 1]], bufb, semb).wait()

            @pl.when(j + 2 < KH)
            def _():
                pltpu.async_copy(g_hbm.at[src_v.at[j + 2]], bufa, sema)

            pltpu.sync_copy(bufb, acc.at[dst_v.at[j + 1]], add=True)
            return carry

        lax.fori_loop(0, KH // 2, step, 0)
    plsc.subcore_barrier()
    pltpu.sync_copy(acc.at[pl.ds(s * RPT, RPT)], p_hbm.at[c, pl.ds(s * RPT, RPT)])


_sc_layer = pl.kernel(
    _sc_layer_body,
    out_type=jax.ShapeDtypeStruct((NC, N_PAD, D), jnp.float32),
    mesh=_MESH,
    scratch_types=[
        pltpu.VMEM((KH, B), jnp.int32),
        pltpu.VMEM((KH, B), jnp.int32),
        pltpu.VMEM((B, D), jnp.float32),
        pltpu.VMEM((B, D), jnp.float32),
        pltpu.VMEM_SHARED((N_PAD, D), jnp.float32),
        pltpu.SemaphoreType.DMA,
        pltpu.SemaphoreType.DMA,
    ],
)


def _tc_first_body(x_ref, w_ref, pt_ref, g_ref, dis_ref):
    deg = pt_ref[0, :, 0:1] + pt_ref[1, :, 0:1] + 1.0
    dv = lax.rsqrt(deg)
    dis_ref[...] = dv
    g_ref[...] = dv * jnp.dot(x_ref[...], w_ref[...],
                              preferred_element_type=jnp.float32)


_tc_first = pl.pallas_call(
    _tc_first_body,
    grid=(G,),
    in_specs=[
        pl.BlockSpec((R, D), lambda i: (i, 0)),
        pl.BlockSpec((D, D), lambda i: (0, 0)),
        pl.BlockSpec((NC, R, 16), lambda i: (0, i, 0)),
    ],
    out_specs=[
        pl.BlockSpec((R, D), lambda i: (i, 0)),
        pl.BlockSpec((R, 1), lambda i: (i, 0)),
    ],
    out_shape=[
        jax.ShapeDtypeStruct((N_PAD, D), jnp.float32),
        jax.ShapeDtypeStruct((N_PAD, 1), jnp.float32),
    ],
)


def _tc_mid_body(p_ref, g_ref, dis_ref, w_ref, b_ref, o_ref):
    dv = dis_ref[...]
    ssum = p_ref[0] + p_ref[1] + g_ref[...]
    a = jnp.maximum(dv * ssum + b_ref[...], 0.0)
    o_ref[...] = dv * jnp.dot(a, w_ref[...], preferred_element_type=jnp.float32)


_tc_mid = pl.pallas_call(
    _tc_mid_body,
    grid=(G,),
    in_specs=[
        pl.BlockSpec((NC, R, D), lambda i: (0, i, 0)),
        pl.BlockSpec((R, D), lambda i: (i, 0)),
        pl.BlockSpec((R, 1), lambda i: (i, 0)),
        pl.BlockSpec((D, D), lambda i: (0, 0)),
        pl.BlockSpec((1, D), lambda i: (0, 0)),
    ],
    out_specs=pl.BlockSpec((R, D), lambda i: (i, 0)),
    out_shape=jax.ShapeDtypeStruct((N_PAD, D), jnp.float32),
)


def _tc_last_body(p_ref, g_ref, dis_ref, b_ref, o_ref):
    o_ref[...] = (dis_ref[...] * (p_ref[0] + p_ref[1] + g_ref[...])
                  + b_ref[...])


_tc_last = pl.pallas_call(
    _tc_last_body,
    grid=(G,),
    in_specs=[
        pl.BlockSpec((NC, R, D), lambda i: (0, i, 0)),
        pl.BlockSpec((R, D), lambda i: (i, 0)),
        pl.BlockSpec((R, 1), lambda i: (i, 0)),
        pl.BlockSpec((1, D), lambda i: (0, 0)),
    ],
    out_specs=pl.BlockSpec((R, D), lambda i: (i, 0)),
    out_shape=jax.ShapeDtypeStruct((N_PAD, D), jnp.float32),
)


def kernel(x, edge_index, W1, b1, W2, b2, W3, b3):
    src = edge_index[0].astype(jnp.int32)
    dst = edge_index[1].astype(jnp.int32)
    pad = N + (jnp.arange(E_PAD - E, dtype=jnp.int32) % (N_PAD - N))
    src_t = jnp.concatenate([src, pad]).reshape(NT, K, B)
    dst_t = jnp.concatenate([dst, pad]).reshape(NT, K, B)
    x_pad = jnp.pad(x, ((0, N_PAD - N), (0, 0)))
    zeros_blk = jnp.zeros((RPT, D), jnp.float32)
    zeros16 = jnp.zeros((RPT, 16), jnp.float32)
    ones16 = jnp.ones((B, 16), jnp.float32)

    degp = _sc_deg(dst_t, ones16, zeros16)
    g1, dis = _tc_first(x_pad, W1, degp)
    p1 = _sc_layer(g1, src_t, dst_t, zeros_blk)
    g2 = _tc_mid(p1, g1, dis, W2, b1.reshape(1, D))
    p2 = _sc_layer(g2, src_t, dst_t, zeros_blk)
    g3 = _tc_mid(p2, g2, dis, W3, b2.reshape(1, D))
    p3 = _sc_layer(g3, src_t, dst_t, zeros_blk)
    out = _tc_last(p3, g3, dis, b3.reshape(1, D))
    return out[:N]
